# Initial kernel scaffold; baseline (speedup 1.0000x reference)
import jax
import jax.numpy as jnp
from jax.experimental import pallas as pl


def kernel(h, edge_index, u, v, edge_attr, eW1, eb1, eW2, eb2, nW1, nb1, nW2, nb2, cW1, cb1, cW2):
    raise NotImplementedError("write your pallas kernel here")


# speedup vs baseline: 1.2411x; 1.2411x over previous
import functools

import jax
import jax.numpy as jnp
from jax.experimental import pallas as pl

N = 29040
E = 929280
D = 64
H = 64

EDGE_BLOCK = 3840
LAT_ROWS = 121
ROW_LEN = 240


def _edge_body(hr, hc, uc, vc, ur, vr, ea, eW1, eb1, eW2, eb2, cW1, cb1, cW2,
               ef_out, wind_out):
    hr = hr[...]
    hc = hc[...]
    uc_ = uc[...]
    vc_ = vc[...]
    ur_ = ur[...]
    vr_ = vr[...]
    col_speed = jnp.sqrt(uc_ * uc_ + vc_ * vc_)
    row_speed = jnp.sqrt(ur_ * ur_ + vr_ * vr_)
    rel_dirt = (uc_ * ur_ + vc_ * vr_) / (col_speed * row_speed)
    x = jnp.concatenate([hr, hc, rel_dirt, col_speed, row_speed, ea[...]], axis=1)
    hid = jax.nn.relu(jnp.dot(x, eW1[...], preferred_element_type=jnp.float32) + eb1[...])
    ef = jax.nn.relu(jnp.dot(hid, eW2[...], preferred_element_type=jnp.float32) + eb2[...])
    ch = jax.nn.relu(jnp.dot(ef, cW1[...], preferred_element_type=jnp.float32) + cb1[...])
    cf = jnp.dot(ch, cW2[...], preferred_element_type=jnp.float32)
    wind_u = cf[:, :11] * uc_
    wind_v = cf[:, 11:] * vc_
    ef_out[...] = ef
    wind_out[...] = jnp.concatenate([wind_u, wind_v], axis=1)


def _node_body(h, agg, sums, cnt, nW1, nb1, nW2, nb2, h_out, u_out, v_out):
    agg_ = agg[...]
    h_ = h[...]
    c = jnp.maximum(cnt[...], 1.0)
    mean = sums[...] / c
    mean = jnp.clip(mean, -100.0, 100.0)
    u_out[...] = mean[:, :11]
    v_out[...] = mean[:, 11:]
    lat = jnp.mean(agg_, axis=0, keepdims=True)
    cat = jnp.concatenate([h_, agg_, jnp.broadcast_to(lat, agg_.shape)], axis=1)
    hid = jax.nn.relu(jnp.dot(cat, nW1[...], preferred_element_type=jnp.float32) + nb1[...])
    h_out[...] = jnp.dot(hid, nW2[...], preferred_element_type=jnp.float32) + nb2[...] + h_


@functools.partial(jax.jit, static_argnames=())
def kernel(h, edge_index, u, v, edge_attr, eW1, eb1, eW2, eb2,
           nW1, nb1, nW2, nb2, cW1, cb1, cW2):
    row = edge_index[0]
    col = edge_index[1]
    hr = h[row]
    hc = h[col]
    uc = u[col]
    vc = v[col]
    ur = u[row]
    vr = v[row]

    grid = E // EDGE_BLOCK
    eb = lambda d: pl.BlockSpec((EDGE_BLOCK, d), lambda i: (i, 0))
    wb = lambda a: pl.BlockSpec(a.shape, lambda i: (0,) * a.ndim)
    ef, wind = pl.pallas_call(
        _edge_body,
        grid=(grid,),
        in_specs=[eb(D), eb(D), eb(11), eb(11), eb(11), eb(11), eb(1),
                  wb(eW1), wb(eb1), wb(eW2), wb(eb2), wb(cW1), wb(cb1), wb(cW2)],
        out_specs=[eb(D), eb(22)],
        out_shape=[jax.ShapeDtypeStruct((E, D), jnp.float32),
                   jax.ShapeDtypeStruct((E, 22), jnp.float32)],
    )(hr, hc, uc, vc, ur, vr, edge_attr, eW1, eb1, eW2, eb2, cW1, cb1, cW2)

    agg = jax.ops.segment_sum(ef, row, num_segments=N)
    sums = jax.ops.segment_sum(wind, row, num_segments=N)
    cnt = jax.ops.segment_sum(jnp.ones((E, 1), jnp.float32), row, num_segments=N)

    nb = lambda d: pl.BlockSpec((ROW_LEN, d), lambda i: (i, 0))
    h_out, agg_u, agg_v = pl.pallas_call(
        _node_body,
        grid=(LAT_ROWS,),
        in_specs=[nb(D), nb(D), nb(22), nb(1),
                  wb(nW1), wb(nb1), wb(nW2), wb(nb2)],
        out_specs=[nb(D), nb(11), nb(11)],
        out_shape=[jax.ShapeDtypeStruct((N, D), jnp.float32),
                   jax.ShapeDtypeStruct((N, 11), jnp.float32),
                   jax.ShapeDtypeStruct((N, 11), jnp.float32)],
    )(h, agg, sums, cnt, nW1, nb1, nW2, nb2)
    return (h_out, agg_u, agg_v)


# speedup vs baseline: 5.8512x; 4.7143x over previous
import functools

import jax
import jax.numpy as jnp
from jax import lax
from jax.experimental import pallas as pl
from jax.experimental.pallas import tpu as pltpu
from jax.experimental.pallas import tpu_sc as plsc

N = 29040
E = 929280
D = 64
H = 64

TW = 128
PW = 48
NC = 2
NS = 16
GCH = 120
EPW = E // (NC * NS)
SPW = E // NS
NZ = 1816

_USE_SC_SCATTER = True

EDGE_BLOCK = 3840
PRE_BLOCK = 2904
LAT_ROWS = 121
ROW_LEN = 240

_MESH = plsc.VectorSubcoreMesh(core_axis_name="c", subcore_axis_name="s")


def _pre_body(h, u, v, W1a, W1b, eb1, t1, t2):
    h_ = h[...]
    pad = jnp.zeros((h_.shape[0], TW - 86), jnp.float32)
    a = jnp.dot(h_, W1a[...], preferred_element_type=jnp.float32) + eb1[...]
    b = jnp.dot(h_, W1b[...], preferred_element_type=jnp.float32)
    t1[...] = jnp.concatenate([a, u[...], v[...], pad], axis=1)
    t2[...] = jnp.concatenate([b, u[...], v[...], pad], axis=1)


def _sc_gather(t1, t2, row, col):
    @functools.partial(
        pl.kernel,
        out_type=[jax.ShapeDtypeStruct((E, TW), jnp.float32),
                  jax.ShapeDtypeStruct((E, TW), jnp.float32)],
        mesh=_MESH,
        scratch_types=[pltpu.VMEM((GCH,), jnp.int32),
                       pltpu.VMEM((GCH,), jnp.int32),
                       pltpu.VMEM((GCH, TW), jnp.float32),
                       pltpu.VMEM((GCH, TW), jnp.float32),
                       pltpu.SemaphoreType.DMA,
                       pltpu.SemaphoreType.DMA],
    )
    def k(t1_hbm, t2_hbm, row_hbm, col_hbm, g1_hbm, g2_hbm,
          idx1, idx2, b1, b2, sem1, sem2):
        wid = lax.axis_index("s") * NC + lax.axis_index("c")
        base = wid * EPW

        @pl.loop(0, EPW // GCH)
        def _(i):
            off = base + i * GCH
            pltpu.sync_copy(row_hbm.at[pl.ds(off, GCH)], idx1)
            pltpu.sync_copy(col_hbm.at[pl.ds(off, GCH)], idx2)
            c1 = pltpu.async_copy(t1_hbm.at[idx1], b1, sem1)
            c2 = pltpu.async_copy(t2_hbm.at[idx2], b2, sem2)
            c1.wait()
            c2.wait()
            pltpu.sync_copy(b1, g1_hbm.at[pl.ds(off, GCH)])
            pltpu.sync_copy(b2, g2_hbm.at[pl.ds(off, GCH)])

    return k(t1, t2, row, col)


def _edge_body(g1, g2, ea, Ww, Wa, eW2, eb2, cW1, cb1, cW2, p):
    g1_ = g1[...]
    g2_ = g2[...]
    ur = g1_[:, 64:75]
    vr = g1_[:, 75:86]
    uc = g2_[:, 64:75]
    vc = g2_[:, 75:86]
    cs = jnp.sqrt(uc * uc + vc * vc)
    rs = jnp.sqrt(ur * ur + vr * vr)
    rd = (uc * ur + vc * vr) / (cs * rs)
    wdiff = jnp.concatenate([rd, cs, rs], axis=1)
    pre = (g1_[:, :64] + g2_[:, :64]
           + jnp.dot(wdiff, Ww[...], preferred_element_type=jnp.float32)
           + ea[...] * Wa[...])
    hid = jax.nn.relu(pre)
    ef = jax.nn.relu(jnp.dot(hid, eW2[...], preferred_element_type=jnp.float32) + eb2[...])
    ch = jax.nn.relu(jnp.dot(ef, cW1[...], preferred_element_type=jnp.float32) + cb1[...])
    cf = jnp.dot(ch, cW2[...], preferred_element_type=jnp.float32)
    wu = cf[:, :11] * uc
    wv = cf[:, 11:] * vc
    ones = jnp.ones((g1_.shape[0], 1), jnp.float32)
    zpad = jnp.zeros((g1_.shape[0], TW - 87), jnp.float32)
    p[...] = jnp.concatenate([ef, wu, wv, ones, zpad], axis=1)


NH = N // NC
ACC_ROWS = NH + 8
SCH = 96
ZR = ACC_ROWS // 8


def _sc_scatter(p, row, zrows):
    @functools.partial(
        pl.kernel,
        out_type=jax.ShapeDtypeStruct((N, TW), jnp.float32),
        mesh=_MESH,
        scratch_types=[pltpu.VMEM((SCH,), jnp.int32),
                       pltpu.VMEM((SCH,), jnp.int32),
                       pltpu.VMEM((SCH, TW), jnp.float32),
                       pltpu.VMEM_SHARED((ACC_ROWS, TW), jnp.float32)],
    )
    def k(p_hbm, row_hbm, z_hbm, o_hbm, idx, idx2, buf, acc):
        c = lax.axis_index("c")
        s = lax.axis_index("s")

        @pl.when(s < 8)
        def _():
            pltpu.sync_copy(z_hbm, acc.at[pl.ds(s * ZR, ZR)])

        plsc.subcore_barrier()
        base = s * SPW
        c_lo = c * NH
        sink = NH + (lax.broadcasted_iota(jnp.int32, (16,), 0) & 7)

        @pl.loop(0, SPW // SCH)
        def _(i):
            off = base + i * SCH
            pltpu.sync_copy(row_hbm.at[pl.ds(off, SCH)], idx)
            for j in range(SCH // 16):
                sl = pl.ds(j * 16, 16)
                l = idx[sl] - c_lo
                ok = (l >= 0) & (l < NH)
                idx2[sl] = jnp.where(ok, l, sink)
            pltpu.sync_copy(p_hbm.at[pl.ds(off, SCH)], buf)
            pltpu.sync_copy(buf, acc.at[idx2], add=True)

        plsc.subcore_barrier()

        @pl.when(s < 7)
        def _():
            pltpu.sync_copy(acc.at[pl.ds(s * ZR, ZR)],
                            o_hbm.at[pl.ds(c_lo + s * ZR, ZR)])

        @pl.when(s == 7)
        def _():
            pltpu.sync_copy(acc.at[pl.ds(7 * ZR, NH - 7 * ZR)],
                            o_hbm.at[pl.ds(c_lo + 7 * ZR, NH - 7 * ZR)])

    return k(p, row, zrows)


def _node_body(h, o, nW1, nb1, nW2, nb2, h_out, u_out, v_out):
    h_ = h[...]
    o_ = o[...]
    agg = o_[:, :64]
    sums = o_[:, 64:86]
    cnt = jnp.maximum(o_[:, 86:87], 1.0)
    mean = jnp.clip(sums / cnt, -100.0, 100.0)
    u_out[...] = mean[:, :11]
    v_out[...] = mean[:, 11:]
    lat = jnp.mean(agg, axis=0, keepdims=True)
    cat = jnp.concatenate([h_, agg, jnp.broadcast_to(lat, agg.shape)], axis=1)
    hid = jax.nn.relu(jnp.dot(cat, nW1[...], preferred_element_type=jnp.float32) + nb1[...])
    h_out[...] = jnp.dot(hid, nW2[...], preferred_element_type=jnp.float32) + nb2[...] + h_


@jax.jit
def kernel(h, edge_index, u, v, edge_attr, eW1, eb1, eW2, eb2,
           nW1, nb1, nW2, nb2, cW1, cb1, cW2):
    row = edge_index[0]
    col = edge_index[1]
    W1a = eW1[0:64]
    W1b = eW1[64:128]
    Ww = eW1[128:161]
    Wa = eW1[161:162]

    wb = lambda a: pl.BlockSpec(a.shape, lambda i: (0,) * a.ndim)
    pb = lambda d: pl.BlockSpec((PRE_BLOCK, d), lambda i: (i, 0))
    t1, t2 = pl.pallas_call(
        _pre_body,
        grid=(N // PRE_BLOCK,),
        in_specs=[pb(D), pb(11), pb(11), wb(W1a), wb(W1b), wb(eb1)],
        out_specs=[pb(TW), pb(TW)],
        out_shape=[jax.ShapeDtypeStruct((N, TW), jnp.float32),
                   jax.ShapeDtypeStruct((N, TW), jnp.float32)],
    )(h, u, v, W1a, W1b, eb1)

    g1, g2 = _sc_gather(t1, t2, row, col)

    ebk = lambda d: pl.BlockSpec((EDGE_BLOCK, d), lambda i: (i, 0))
    p = pl.pallas_call(
        _edge_body,
        grid=(E // EDGE_BLOCK,),
        in_specs=[ebk(TW), ebk(TW), ebk(1),
                  wb(Ww), wb(Wa), wb(eW2), wb(eb2), wb(cW1), wb(cb1), wb(cW2)],
        out_specs=[ebk(TW)],
        out_shape=[jax.ShapeDtypeStruct((E, TW), jnp.float32)],
    )(g1, g2, edge_attr, Ww, Wa, eW2, eb2, cW1, cb1, cW2)[0]

    if _USE_SC_SCATTER:
        zrows = jnp.zeros((ZR, TW), jnp.float32)
        o = _sc_scatter(p, row, zrows)
    else:
        o = jax.ops.segment_sum(p, row, num_segments=N)

    nbk = lambda d: pl.BlockSpec((ROW_LEN, d), lambda i: (i, 0))
    h_out, agg_u, agg_v = pl.pallas_call(
        _node_body,
        grid=(LAT_ROWS,),
        in_specs=[nbk(D), nbk(TW),
                  wb(nW1), wb(nb1), wb(nW2), wb(nb2)],
        out_specs=[nbk(D), nbk(11), nbk(11)],
        out_shape=[jax.ShapeDtypeStruct((N, D), jnp.float32),
                   jax.ShapeDtypeStruct((N, 11), jnp.float32),
                   jax.ShapeDtypeStruct((N, 11), jnp.float32)],
    )(h, o, nW1, nb1, nW2, nb2)
    return (h_out, agg_u, agg_v)


# speedup vs baseline: 7.3977x; 1.2643x over previous
import functools

import jax
import jax.numpy as jnp
from jax import lax
from jax.experimental import pallas as pl
from jax.experimental.pallas import tpu as pltpu
from jax.experimental.pallas import tpu_sc as plsc

N = 29040
E = 929280
D = 64
H = 64

TW = 128
PW = 48
NC = 2
NS = 16
GCH = 120
EPW = E // (NC * NS)
SPW = E // NS
NZ = 1816

K_CHUNKS = 2

EDGE_BLOCK = 3840
PRE_BLOCK = 2904
LAT_ROWS = 121
ROW_LEN = 240

_MESH = plsc.VectorSubcoreMesh(core_axis_name="c", subcore_axis_name="s")


def _pre_body(h, u, v, W1a, W1b, eb1, t1, t2):
    h_ = h[...]
    pad = jnp.zeros((h_.shape[0], TW - 86), jnp.float32)
    a = jnp.dot(h_, W1a[...], preferred_element_type=jnp.float32) + eb1[...]
    b = jnp.dot(h_, W1b[...], preferred_element_type=jnp.float32)
    t1[...] = jnp.concatenate([a, u[...], v[...], pad], axis=1)
    t2[...] = jnp.concatenate([b, u[...], v[...], pad], axis=1)


def _sc_gather(t1, t2, rowc, colc, n_edges):
    epw = n_edges // (NC * NS)

    @functools.partial(
        pl.kernel,
        out_type=[jax.ShapeDtypeStruct((n_edges, TW), jnp.float32),
                  jax.ShapeDtypeStruct((n_edges, TW), jnp.float32)],
        mesh=_MESH,
        scratch_types=[pltpu.VMEM((GCH,), jnp.int32),
                       pltpu.VMEM((GCH,), jnp.int32),
                       pltpu.VMEM((GCH, TW), jnp.float32),
                       pltpu.VMEM((GCH, TW), jnp.float32),
                       pltpu.SemaphoreType.DMA,
                       pltpu.SemaphoreType.DMA],
    )
    def k(t1_hbm, t2_hbm, row_hbm, col_hbm, g1_hbm, g2_hbm,
          idx1, idx2, b1, b2, sem1, sem2):
        wid = lax.axis_index("s") * NC + lax.axis_index("c")
        base = wid * epw

        @pl.loop(0, epw // GCH)
        def _(i):
            off = base + i * GCH
            pltpu.sync_copy(row_hbm.at[pl.ds(off, GCH)], idx1)
            pltpu.sync_copy(col_hbm.at[pl.ds(off, GCH)], idx2)
            c1 = pltpu.async_copy(t1_hbm.at[idx1], b1, sem1)
            c2 = pltpu.async_copy(t2_hbm.at[idx2], b2, sem2)
            c1.wait()
            c2.wait()
            pltpu.sync_copy(b1, g1_hbm.at[pl.ds(off, GCH)])
            pltpu.sync_copy(b2, g2_hbm.at[pl.ds(off, GCH)])

    return k(t1, t2, rowc, colc)


def _edge_body(g1, g2, ea, Ww, Wa, eW2, eb2, cW1, cb1, cW2, p):
    g1_ = g1[...]
    g2_ = g2[...]
    ur = g1_[:, 64:75]
    vr = g1_[:, 75:86]
    uc = g2_[:, 64:75]
    vc = g2_[:, 75:86]
    cs = jnp.sqrt(uc * uc + vc * vc)
    rs = jnp.sqrt(ur * ur + vr * vr)
    rd = (uc * ur + vc * vr) / (cs * rs)
    bf = jnp.bfloat16
    wdiff = jnp.concatenate([rd, cs, rs], axis=1).astype(bf)
    pre = (g1_[:, :64] + g2_[:, :64]
           + jnp.dot(wdiff, Ww[...].astype(bf), preferred_element_type=jnp.float32)
           + ea[...] * Wa[...])
    hid = jax.nn.relu(pre).astype(bf)
    ef = jax.nn.relu(jnp.dot(hid, eW2[...].astype(bf), preferred_element_type=jnp.float32) + eb2[...])
    ch = jax.nn.relu(jnp.dot(ef.astype(bf), cW1[...].astype(bf), preferred_element_type=jnp.float32) + cb1[...]).astype(bf)
    cf = jnp.dot(ch, cW2[...].astype(bf), preferred_element_type=jnp.float32)
    wu = cf[:, :11] * uc
    wv = cf[:, 11:] * vc
    ones = jnp.ones((g1_.shape[0], 1), jnp.float32)
    zpad = jnp.zeros((g1_.shape[0], TW - 87), jnp.float32)
    p[...] = jnp.concatenate([ef, wu, wv, ones, zpad], axis=1)


NH = N // NC
ACC_ROWS = NH + 8
SCH = 80
ZR = 1816


def _sc_scatter(p, rowc, o_prev, n_edges):
    epw = n_edges // NS

    @functools.partial(
        pl.kernel,
        out_type=jax.ShapeDtypeStruct((N, TW), jnp.float32),
        mesh=_MESH,
        scratch_types=[pltpu.VMEM((SCH,), jnp.int32),
                       pltpu.VMEM((SCH,), jnp.int32),
                       pltpu.VMEM((SCH, TW), jnp.float32),
                       pltpu.VMEM_SHARED((ACC_ROWS, TW), jnp.float32)],
    )
    def k(p_hbm, row_hbm, oprev_hbm, o_hbm, idx, idx2, buf, acc):
        c = lax.axis_index("c")
        s = lax.axis_index("s")
        c_lo = c * NH

        @pl.when(s < 7)
        def _():
            pltpu.sync_copy(oprev_hbm.at[pl.ds(c_lo + s * ZR, ZR)],
                            acc.at[pl.ds(s * ZR, ZR)])

        @pl.when(s == 7)
        def _():
            pltpu.sync_copy(oprev_hbm.at[pl.ds(c_lo + 7 * ZR, NH - 7 * ZR)],
                            acc.at[pl.ds(7 * ZR, NH - 7 * ZR)])

        plsc.subcore_barrier()
        base = s * epw
        sink = NH + (lax.broadcasted_iota(jnp.int32, (16,), 0) & 7)

        @pl.loop(0, epw // SCH)
        def _(i):
            off = base + i * SCH
            pltpu.sync_copy(row_hbm.at[pl.ds(off, SCH)], idx)
            for j in range(SCH // 16):
                sl = pl.ds(j * 16, 16)
                l = idx[sl] - c_lo
                ok = (l >= 0) & (l < NH)
                idx2[sl] = jnp.where(ok, l, sink)
            pltpu.sync_copy(p_hbm.at[pl.ds(off, SCH)], buf)
            pltpu.sync_copy(buf, acc.at[idx2], add=True)

        plsc.subcore_barrier()

        @pl.when(s < 7)
        def _():
            pltpu.sync_copy(acc.at[pl.ds(s * ZR, ZR)],
                            o_hbm.at[pl.ds(c_lo + s * ZR, ZR)])

        @pl.when(s == 7)
        def _():
            pltpu.sync_copy(acc.at[pl.ds(7 * ZR, NH - 7 * ZR)],
                            o_hbm.at[pl.ds(c_lo + 7 * ZR, NH - 7 * ZR)])

    return k(p, rowc, o_prev)


def _node_body(h, o, nW1, nb1, nW2, nb2, h_out, u_out, v_out):
    h_ = h[...]
    o_ = o[...]
    agg = o_[:, :64]
    sums = o_[:, 64:86]
    cnt = jnp.maximum(o_[:, 86:87], 1.0)
    mean = jnp.clip(sums / cnt, -100.0, 100.0)
    u_out[...] = mean[:, :11]
    v_out[...] = mean[:, 11:]
    lat = jnp.mean(agg, axis=0, keepdims=True)
    cat = jnp.concatenate([h_, agg, jnp.broadcast_to(lat, agg.shape)], axis=1)
    hid = jax.nn.relu(jnp.dot(cat, nW1[...], preferred_element_type=jnp.float32) + nb1[...])
    h_out[...] = jnp.dot(hid, nW2[...], preferred_element_type=jnp.float32) + nb2[...] + h_


@jax.jit
def kernel(h, edge_index, u, v, edge_attr, eW1, eb1, eW2, eb2,
           nW1, nb1, nW2, nb2, cW1, cb1, cW2):
    row = edge_index[0]
    col = edge_index[1]
    W1a = eW1[0:64]
    W1b = eW1[64:128]
    Ww = eW1[128:161]
    Wa = eW1[161:162]

    wb = lambda a: pl.BlockSpec(a.shape, lambda i: (0,) * a.ndim)
    pb = lambda d: pl.BlockSpec((PRE_BLOCK, d), lambda i: (i, 0))
    t1, t2 = pl.pallas_call(
        _pre_body,
        grid=(N // PRE_BLOCK,),
        in_specs=[pb(D), pb(11), pb(11), wb(W1a), wb(W1b), wb(eb1)],
        out_specs=[pb(TW), pb(TW)],
        out_shape=[jax.ShapeDtypeStruct((N, TW), jnp.float32),
                   jax.ShapeDtypeStruct((N, TW), jnp.float32)],
    )(h, u, v, W1a, W1b, eb1)

    ebk = lambda d: pl.BlockSpec((EDGE_BLOCK, d), lambda i: (i, 0))
    eh = E // K_CHUNKS
    o = jnp.zeros((N, TW), jnp.float32)
    for kc in range(K_CHUNKS):
        sl = slice(kc * eh, (kc + 1) * eh)
        g1, g2 = _sc_gather(t1, t2, row[sl], col[sl], eh)
        p = pl.pallas_call(
            _edge_body,
            grid=(eh // EDGE_BLOCK,),
            in_specs=[ebk(TW), ebk(TW), ebk(1),
                      wb(Ww), wb(Wa), wb(eW2), wb(eb2), wb(cW1), wb(cb1), wb(cW2)],
            out_specs=[ebk(TW)],
            out_shape=[jax.ShapeDtypeStruct((eh, TW), jnp.float32)],
        )(g1, g2, edge_attr[sl], Ww, Wa, eW2, eb2, cW1, cb1, cW2)[0]
        o = _sc_scatter(p, row[sl], o, eh)

    nbk = lambda d: pl.BlockSpec((ROW_LEN, d), lambda i: (i, 0))
    h_out, agg_u, agg_v = pl.pallas_call(
        _node_body,
        grid=(LAT_ROWS,),
        in_specs=[nbk(D), nbk(TW),
                  wb(nW1), wb(nb1), wb(nW2), wb(nb2)],
        out_specs=[nbk(D), nbk(11), nbk(11)],
        out_shape=[jax.ShapeDtypeStruct((N, D), jnp.float32),
                   jax.ShapeDtypeStruct((N, 11), jnp.float32),
                   jax.ShapeDtypeStruct((N, 11), jnp.float32)],
    )(h, o, nW1, nb1, nW2, nb2)
    return (h_out, agg_u, agg_v)


# speedup vs baseline: 7.5908x; 1.0261x over previous
import functools

import jax
import jax.numpy as jnp
from jax import lax
from jax.experimental import pallas as pl
from jax.experimental.pallas import tpu as pltpu
from jax.experimental.pallas import tpu_sc as plsc

N = 29040
E = 929280
D = 64
H = 64

TW = 128
PW = 48
NC = 2
NS = 16
GCH = 120
EPW = E // (NC * NS)
SPW = E // NS
NZ = 1816

K_CHUNKS = 2

EDGE_BLOCK = 3840
PRE_BLOCK = 2904
LAT_ROWS = 121
ROW_LEN = 240

_MESH = plsc.VectorSubcoreMesh(core_axis_name="c", subcore_axis_name="s")


def _pre_body(h, u, v, W1a, W1b, eb1, t1, t2):
    h_ = h[...]
    pad = jnp.zeros((h_.shape[0], TW - 86), jnp.float32)
    a = jnp.dot(h_, W1a[...], preferred_element_type=jnp.float32) + eb1[...]
    b = jnp.dot(h_, W1b[...], preferred_element_type=jnp.float32)
    t1[...] = jnp.concatenate([a, u[...], v[...], pad], axis=1)
    t2[...] = jnp.concatenate([b, u[...], v[...], pad], axis=1)


def _sc_gather(t1, t2, rowc, colc, n_edges):
    epw = n_edges // (NC * NS)

    @functools.partial(
        pl.kernel,
        out_type=[jax.ShapeDtypeStruct((n_edges, TW), jnp.float32),
                  jax.ShapeDtypeStruct((n_edges, TW), jnp.float32)],
        mesh=_MESH,
        scratch_types=[pltpu.VMEM((GCH,), jnp.int32),
                       pltpu.VMEM((GCH,), jnp.int32),
                       pltpu.VMEM((GCH, TW), jnp.float32),
                       pltpu.VMEM((GCH, TW), jnp.float32),
                       pltpu.SemaphoreType.DMA,
                       pltpu.SemaphoreType.DMA],
    )
    def k(t1_hbm, t2_hbm, row_hbm, col_hbm, g1_hbm, g2_hbm,
          idx1, idx2, b1, b2, sem1, sem2):
        wid = lax.axis_index("s") * NC + lax.axis_index("c")
        base = wid * epw

        @pl.loop(0, epw // GCH)
        def _(i):
            off = base + i * GCH
            pltpu.sync_copy(row_hbm.at[pl.ds(off, GCH)], idx1)
            pltpu.sync_copy(col_hbm.at[pl.ds(off, GCH)], idx2)
            c1 = pltpu.async_copy(t1_hbm.at[idx1], b1, sem1)
            c2 = pltpu.async_copy(t2_hbm.at[idx2], b2, sem2)
            c1.wait()
            c2.wait()
            pltpu.sync_copy(b1, g1_hbm.at[pl.ds(off, GCH)])
            pltpu.sync_copy(b2, g2_hbm.at[pl.ds(off, GCH)])

    return k(t1, t2, rowc, colc)


def _edge_body(g1, g2, ea, Ww, Wa, eW2, eb2, cW1, cb1, cW2, p):
    g1_ = g1[...]
    g2_ = g2[...]
    ur = g1_[:, 64:75]
    vr = g1_[:, 75:86]
    uc = g2_[:, 64:75]
    vc = g2_[:, 75:86]
    cs = jnp.sqrt(uc * uc + vc * vc)
    rs = jnp.sqrt(ur * ur + vr * vr)
    rd = (uc * ur + vc * vr) / (cs * rs)
    bf = jnp.bfloat16
    wdiff = jnp.concatenate([rd, cs, rs], axis=1).astype(bf)
    pre = (g1_[:, :64] + g2_[:, :64]
           + jnp.dot(wdiff, Ww[...].astype(bf), preferred_element_type=jnp.float32)
           + ea[...] * Wa[...])
    hid = jax.nn.relu(pre).astype(bf)
    ef = jax.nn.relu(jnp.dot(hid, eW2[...].astype(bf), preferred_element_type=jnp.float32) + eb2[...])
    ch = jax.nn.relu(jnp.dot(ef.astype(bf), cW1[...].astype(bf), preferred_element_type=jnp.float32) + cb1[...]).astype(bf)
    cf = jnp.dot(ch, cW2[...].astype(bf), preferred_element_type=jnp.float32)
    wu = cf[:, :11] * uc
    wv = cf[:, 11:] * vc
    ones = jnp.ones((g1_.shape[0], 1), jnp.float32)
    zpad = jnp.zeros((g1_.shape[0], TW - 87), jnp.float32)
    p[...] = jnp.concatenate([ef, wu, wv, ones, zpad], axis=1)


NH = N // NC
ACC_ROWS = NH + 8
SCH = 88
ZR = 1816


def _sc_scatter(p, row0c, row1c, o_prev, n_edges):
    epw = n_edges // NS

    @functools.partial(
        pl.kernel,
        out_type=jax.ShapeDtypeStruct((N, TW), jnp.float32),
        mesh=_MESH,
        scratch_types=[pltpu.VMEM((SCH,), jnp.int32),
                       pltpu.VMEM((SCH, TW), jnp.float32),
                       pltpu.VMEM_SHARED((ACC_ROWS, TW), jnp.float32)],

    )
    def k(p_hbm, row0_hbm, row1_hbm, oprev_hbm, o_hbm, idx, buf, acc):
        c = lax.axis_index("c")
        s = lax.axis_index("s")
        c_lo = c * NH

        @pl.when(s < 7)
        def _():
            pltpu.sync_copy(oprev_hbm.at[pl.ds(c_lo + s * ZR, ZR)],
                            acc.at[pl.ds(s * ZR, ZR)])

        @pl.when(s == 7)
        def _():
            pltpu.sync_copy(oprev_hbm.at[pl.ds(c_lo + 7 * ZR, NH - 7 * ZR)],
                            acc.at[pl.ds(7 * ZR, NH - 7 * ZR)])

        plsc.subcore_barrier()
        base = s * epw

        @pl.loop(0, epw // SCH)
        def _(i):
            off = base + i * SCH
            @pl.when(c == 0)
            def _():
                pltpu.sync_copy(row0_hbm.at[pl.ds(off, SCH)], idx)

            @pl.when(c == 1)
            def _():
                pltpu.sync_copy(row1_hbm.at[pl.ds(off, SCH)], idx)

            pltpu.sync_copy(p_hbm.at[pl.ds(off, SCH)], buf)
            pltpu.sync_copy(buf, acc.at[idx], add=True)

        plsc.subcore_barrier()

        @pl.when(s < 7)
        def _():
            pltpu.sync_copy(acc.at[pl.ds(s * ZR, ZR)],
                            o_hbm.at[pl.ds(c_lo + s * ZR, ZR)])

        @pl.when(s == 7)
        def _():
            pltpu.sync_copy(acc.at[pl.ds(7 * ZR, NH - 7 * ZR)],
                            o_hbm.at[pl.ds(c_lo + 7 * ZR, NH - 7 * ZR)])

    return k(p, row0c, row1c, o_prev)


def _node_body(h, o, nW1, nb1, nW2, nb2, h_out, u_out, v_out):
    h_ = h[...]
    o_ = o[...]
    agg = o_[:, :64]
    sums = o_[:, 64:86]
    cnt = jnp.maximum(o_[:, 86:87], 1.0)
    mean = jnp.clip(sums / cnt, -100.0, 100.0)
    u_out[...] = mean[:, :11]
    v_out[...] = mean[:, 11:]
    lat = jnp.mean(agg, axis=0, keepdims=True)
    cat = jnp.concatenate([h_, agg, jnp.broadcast_to(lat, agg.shape)], axis=1)
    hid = jax.nn.relu(jnp.dot(cat, nW1[...], preferred_element_type=jnp.float32) + nb1[...])
    h_out[...] = jnp.dot(hid, nW2[...], preferred_element_type=jnp.float32) + nb2[...] + h_


@jax.jit
def kernel(h, edge_index, u, v, edge_attr, eW1, eb1, eW2, eb2,
           nW1, nb1, nW2, nb2, cW1, cb1, cW2):
    row = edge_index[0]
    col = edge_index[1]
    W1a = eW1[0:64]
    W1b = eW1[64:128]
    Ww = eW1[128:161]
    Wa = eW1[161:162]

    wb = lambda a: pl.BlockSpec(a.shape, lambda i: (0,) * a.ndim)
    pb = lambda d: pl.BlockSpec((PRE_BLOCK, d), lambda i: (i, 0))
    t1, t2 = pl.pallas_call(
        _pre_body,
        grid=(N // PRE_BLOCK,),
        in_specs=[pb(D), pb(11), pb(11), wb(W1a), wb(W1b), wb(eb1)],
        out_specs=[pb(TW), pb(TW)],
        out_shape=[jax.ShapeDtypeStruct((N, TW), jnp.float32),
                   jax.ShapeDtypeStruct((N, TW), jnp.float32)],
    )(h, u, v, W1a, W1b, eb1)

    sink = NH + (jnp.arange(E, dtype=jnp.int32) & 7)
    row0 = jnp.where(row < NH, row, sink)
    row1 = jnp.where(row >= NH, row - NH, sink)

    ebk = lambda d: pl.BlockSpec((EDGE_BLOCK, d), lambda i: (i, 0))
    eh = E // K_CHUNKS
    o = jnp.zeros((N, TW), jnp.float32)
    for kc in range(K_CHUNKS):
        sl = slice(kc * eh, (kc + 1) * eh)
        g1, g2 = _sc_gather(t1, t2, row[sl], col[sl], eh)
        p = pl.pallas_call(
            _edge_body,
            grid=(eh // EDGE_BLOCK,),
            in_specs=[ebk(TW), ebk(TW), ebk(1),
                      wb(Ww), wb(Wa), wb(eW2), wb(eb2), wb(cW1), wb(cb1), wb(cW2)],
            out_specs=[ebk(TW)],
            out_shape=[jax.ShapeDtypeStruct((eh, TW), jnp.float32)],
        )(g1, g2, edge_attr[sl], Ww, Wa, eW2, eb2, cW1, cb1, cW2)[0]
        o = _sc_scatter(p, row0[sl], row1[sl], o, eh)

    nbk = lambda d: pl.BlockSpec((ROW_LEN, d), lambda i: (i, 0))
    h_out, agg_u, agg_v = pl.pallas_call(
        _node_body,
        grid=(LAT_ROWS,),
        in_specs=[nbk(D), nbk(TW),
                  wb(nW1), wb(nb1), wb(nW2), wb(nb2)],
        out_specs=[nbk(D), nbk(11), nbk(11)],
        out_shape=[jax.ShapeDtypeStruct((N, D), jnp.float32),
                   jax.ShapeDtypeStruct((N, 11), jnp.float32),
                   jax.ShapeDtypeStruct((N, 11), jnp.float32)],
    )(h, o, nW1, nb1, nW2, nb2)
    return (h_out, agg_u, agg_v)


# speedup vs baseline: 7.8638x; 1.0360x over previous
import functools

import jax
import jax.numpy as jnp
from jax import lax
from jax.experimental import pallas as pl
from jax.experimental.pallas import tpu as pltpu
from jax.experimental.pallas import tpu_sc as plsc

N = 29040
E = 929280
D = 64
H = 64

TW = 128
PW = 48
NC = 2
NS = 16
GCH = 120
EPW = E // (NC * NS)
SPW = E // NS
NZ = 1816

K_CHUNKS = 2

EDGE_BLOCK = 3840
PRE_BLOCK = 2904
LAT_ROWS = 121
ROW_LEN = 240

_MESH = plsc.VectorSubcoreMesh(core_axis_name="c", subcore_axis_name="s")


def _pre_body(h, u, v, W1a, W1b, eb1, t1, t2):
    h_ = h[...]
    pad = jnp.zeros((h_.shape[0], TW - 86), jnp.float32)
    a = jnp.dot(h_, W1a[...], preferred_element_type=jnp.float32) + eb1[...]
    b = jnp.dot(h_, W1b[...], preferred_element_type=jnp.float32)
    t1[...] = jnp.concatenate([a, u[...], v[...], pad], axis=1)
    t2[...] = jnp.concatenate([b, u[...], v[...], pad], axis=1)


def _sc_gather(t1, t2, rowc, colc, n_edges):
    epw = n_edges // (NC * NS)

    @functools.partial(
        pl.kernel,
        out_type=[jax.ShapeDtypeStruct((n_edges, TW), jnp.float32),
                  jax.ShapeDtypeStruct((n_edges, TW), jnp.float32)],
        mesh=_MESH,
        scratch_types=[pltpu.VMEM((GCH,), jnp.int32),
                       pltpu.VMEM((GCH,), jnp.int32),
                       pltpu.VMEM((GCH,), jnp.int32),
                       pltpu.VMEM((GCH,), jnp.int32),
                       pltpu.VMEM((GCH, TW), jnp.float32),
                       pltpu.VMEM((GCH, TW), jnp.float32),
                       pltpu.VMEM((GCH, TW), jnp.float32),
                       pltpu.VMEM((GCH, TW), jnp.float32),
                       pltpu.SemaphoreType.DMA,
                       pltpu.SemaphoreType.DMA,
                       pltpu.SemaphoreType.DMA,
                       pltpu.SemaphoreType.DMA,
                       pltpu.SemaphoreType.DMA,
                       pltpu.SemaphoreType.DMA,
                       pltpu.SemaphoreType.DMA,
                       pltpu.SemaphoreType.DMA],
    )
    def k(t1_hbm, t2_hbm, row_hbm, col_hbm, g1_hbm, g2_hbm,
          i1a, i2a, i1b, i2b, b1a, b2a, b1b, b2b,
          s1, s2, s3, s4, s5, s6, s7, s8):
        wid = lax.axis_index("s") * NC + lax.axis_index("c")
        base = wid * epw

        @pl.loop(0, epw // (2 * GCH))
        def _(j):
            offa = base + 2 * j * GCH
            offb = offa + GCH
            pltpu.sync_copy(row_hbm.at[pl.ds(offa, GCH)], i1a)
            pltpu.sync_copy(col_hbm.at[pl.ds(offa, GCH)], i2a)
            ga1 = pltpu.async_copy(t1_hbm.at[i1a], b1a, s1)
            ga2 = pltpu.async_copy(t2_hbm.at[i2a], b2a, s2)
            pltpu.sync_copy(row_hbm.at[pl.ds(offb, GCH)], i1b)
            pltpu.sync_copy(col_hbm.at[pl.ds(offb, GCH)], i2b)
            gb1 = pltpu.async_copy(t1_hbm.at[i1b], b1b, s3)
            gb2 = pltpu.async_copy(t2_hbm.at[i2b], b2b, s4)
            ga1.wait()
            ga2.wait()
            wa1 = pltpu.async_copy(b1a, g1_hbm.at[pl.ds(offa, GCH)], s5)
            wa2 = pltpu.async_copy(b2a, g2_hbm.at[pl.ds(offa, GCH)], s6)
            gb1.wait()
            gb2.wait()
            wb1 = pltpu.async_copy(b1b, g1_hbm.at[pl.ds(offb, GCH)], s7)
            wb2 = pltpu.async_copy(b2b, g2_hbm.at[pl.ds(offb, GCH)], s8)
            wa1.wait()
            wa2.wait()
            wb1.wait()
            wb2.wait()

        if (epw // GCH) % 2 == 1:
            off = base + (epw // GCH - 1) * GCH
            pltpu.sync_copy(row_hbm.at[pl.ds(off, GCH)], i1a)
            pltpu.sync_copy(col_hbm.at[pl.ds(off, GCH)], i2a)
            g1 = pltpu.async_copy(t1_hbm.at[i1a], b1a, s1)
            g2 = pltpu.async_copy(t2_hbm.at[i2a], b2a, s2)
            g1.wait()
            g2.wait()
            pltpu.sync_copy(b1a, g1_hbm.at[pl.ds(off, GCH)])
            pltpu.sync_copy(b2a, g2_hbm.at[pl.ds(off, GCH)])

    return k(t1, t2, rowc, colc)


def _edge_body(g1, g2, ea, Ww, Wa, eW2, eb2, cW1, cb1, cW2, p):
    g1_ = g1[...]
    g2_ = g2[...]
    ur = g1_[:, 64:75]
    vr = g1_[:, 75:86]
    uc = g2_[:, 64:75]
    vc = g2_[:, 75:86]
    cs = jnp.sqrt(uc * uc + vc * vc)
    rs = jnp.sqrt(ur * ur + vr * vr)
    rd = (uc * ur + vc * vr) / (cs * rs)
    bf = jnp.bfloat16
    wdiff = jnp.concatenate([rd, cs, rs], axis=1).astype(bf)
    pre = (g1_[:, :64] + g2_[:, :64]
           + jnp.dot(wdiff, Ww[...].astype(bf), preferred_element_type=jnp.float32)
           + ea[...] * Wa[...])
    hid = jax.nn.relu(pre).astype(bf)
    ef = jax.nn.relu(jnp.dot(hid, eW2[...].astype(bf), preferred_element_type=jnp.float32) + eb2[...])
    ch = jax.nn.relu(jnp.dot(ef.astype(bf), cW1[...].astype(bf), preferred_element_type=jnp.float32) + cb1[...]).astype(bf)
    cf = jnp.dot(ch, cW2[...].astype(bf), preferred_element_type=jnp.float32)
    wu = cf[:, :11] * uc
    wv = cf[:, 11:] * vc
    ones = jnp.ones((g1_.shape[0], 1), jnp.float32)
    zpad = jnp.zeros((g1_.shape[0], TW - 87), jnp.float32)
    p[...] = jnp.concatenate([ef, wu, wv, ones, zpad], axis=1)


NH = N // NC
ACC_ROWS = NH + 8
SCH = 88
ZR = 1816


def _sc_scatter(p, row0c, row1c, o_prev, n_edges):
    epw = n_edges // NS

    @functools.partial(
        pl.kernel,
        out_type=jax.ShapeDtypeStruct((N, TW), jnp.float32),
        mesh=_MESH,
        scratch_types=[pltpu.VMEM((SCH,), jnp.int32),
                       pltpu.VMEM((SCH, TW), jnp.float32),
                       pltpu.VMEM_SHARED((ACC_ROWS, TW), jnp.float32)],

    )
    def k(p_hbm, row0_hbm, row1_hbm, oprev_hbm, o_hbm, idx, buf, acc):
        c = lax.axis_index("c")
        s = lax.axis_index("s")
        c_lo = c * NH

        @pl.when(s < 7)
        def _():
            pltpu.sync_copy(oprev_hbm.at[pl.ds(c_lo + s * ZR, ZR)],
                            acc.at[pl.ds(s * ZR, ZR)])

        @pl.when(s == 7)
        def _():
            pltpu.sync_copy(oprev_hbm.at[pl.ds(c_lo + 7 * ZR, NH - 7 * ZR)],
                            acc.at[pl.ds(7 * ZR, NH - 7 * ZR)])

        plsc.subcore_barrier()
        base = s * epw

        @pl.loop(0, epw // SCH)
        def _(i):
            off = base + i * SCH
            @pl.when(c == 0)
            def _():
                pltpu.sync_copy(row0_hbm.at[pl.ds(off, SCH)], idx)

            @pl.when(c == 1)
            def _():
                pltpu.sync_copy(row1_hbm.at[pl.ds(off, SCH)], idx)

            pltpu.sync_copy(p_hbm.at[pl.ds(off, SCH)], buf)
            pltpu.sync_copy(buf, acc.at[idx], add=True)

        plsc.subcore_barrier()

        @pl.when(s < 7)
        def _():
            pltpu.sync_copy(acc.at[pl.ds(s * ZR, ZR)],
                            o_hbm.at[pl.ds(c_lo + s * ZR, ZR)])

        @pl.when(s == 7)
        def _():
            pltpu.sync_copy(acc.at[pl.ds(7 * ZR, NH - 7 * ZR)],
                            o_hbm.at[pl.ds(c_lo + 7 * ZR, NH - 7 * ZR)])

    return k(p, row0c, row1c, o_prev)


def _node_body(h, o, nW1, nb1, nW2, nb2, h_out, u_out, v_out):
    h_ = h[...]
    o_ = o[...]
    agg = o_[:, :64]
    sums = o_[:, 64:86]
    cnt = jnp.maximum(o_[:, 86:87], 1.0)
    mean = jnp.clip(sums / cnt, -100.0, 100.0)
    u_out[...] = mean[:, :11]
    v_out[...] = mean[:, 11:]
    lat = jnp.mean(agg, axis=0, keepdims=True)
    cat = jnp.concatenate([h_, agg, jnp.broadcast_to(lat, agg.shape)], axis=1)
    hid = jax.nn.relu(jnp.dot(cat, nW1[...], preferred_element_type=jnp.float32) + nb1[...])
    h_out[...] = jnp.dot(hid, nW2[...], preferred_element_type=jnp.float32) + nb2[...] + h_


@jax.jit
def kernel(h, edge_index, u, v, edge_attr, eW1, eb1, eW2, eb2,
           nW1, nb1, nW2, nb2, cW1, cb1, cW2):
    row = edge_index[0]
    col = edge_index[1]
    W1a = eW1[0:64]
    W1b = eW1[64:128]
    Ww = eW1[128:161]
    Wa = eW1[161:162]

    wb = lambda a: pl.BlockSpec(a.shape, lambda i: (0,) * a.ndim)
    pb = lambda d: pl.BlockSpec((PRE_BLOCK, d), lambda i: (i, 0))
    t1, t2 = pl.pallas_call(
        _pre_body,
        grid=(N // PRE_BLOCK,),
        in_specs=[pb(D), pb(11), pb(11), wb(W1a), wb(W1b), wb(eb1)],
        out_specs=[pb(TW), pb(TW)],
        out_shape=[jax.ShapeDtypeStruct((N, TW), jnp.float32),
                   jax.ShapeDtypeStruct((N, TW), jnp.float32)],
    )(h, u, v, W1a, W1b, eb1)

    sink = NH + (jnp.arange(E, dtype=jnp.int32) & 7)
    row0 = jnp.where(row < NH, row, sink)
    row1 = jnp.where(row >= NH, row - NH, sink)

    ebk = lambda d: pl.BlockSpec((EDGE_BLOCK, d), lambda i: (i, 0))
    eh = E // K_CHUNKS
    o = jnp.zeros((N, TW), jnp.float32)
    for kc in range(K_CHUNKS):
        sl = slice(kc * eh, (kc + 1) * eh)
        g1, g2 = _sc_gather(t1, t2, row[sl], col[sl], eh)
        p = pl.pallas_call(
            _edge_body,
            grid=(eh // EDGE_BLOCK,),
            in_specs=[ebk(TW), ebk(TW), ebk(1),
                      wb(Ww), wb(Wa), wb(eW2), wb(eb2), wb(cW1), wb(cb1), wb(cW2)],
            out_specs=[ebk(TW)],
            out_shape=[jax.ShapeDtypeStruct((eh, TW), jnp.float32)],
        )(g1, g2, edge_attr[sl], Ww, Wa, eW2, eb2, cW1, cb1, cW2)[0]
        o = _sc_scatter(p, row0[sl], row1[sl], o, eh)

    nbk = lambda d: pl.BlockSpec((ROW_LEN, d), lambda i: (i, 0))
    h_out, agg_u, agg_v = pl.pallas_call(
        _node_body,
        grid=(LAT_ROWS,),
        in_specs=[nbk(D), nbk(TW),
                  wb(nW1), wb(nb1), wb(nW2), wb(nb2)],
        out_specs=[nbk(D), nbk(11), nbk(11)],
        out_shape=[jax.ShapeDtypeStruct((N, D), jnp.float32),
                   jax.ShapeDtypeStruct((N, 11), jnp.float32),
                   jax.ShapeDtypeStruct((N, 11), jnp.float32)],
    )(h, o, nW1, nb1, nW2, nb2)
    return (h_out, agg_u, agg_v)


# speedup vs baseline: 8.6222x; 1.0964x over previous
import functools

import jax
import jax.numpy as jnp
from jax import lax
from jax.experimental import pallas as pl
from jax.experimental.pallas import tpu as pltpu
from jax.experimental.pallas import tpu_sc as plsc

N = 29040
E = 929280
D = 64
H = 64

TW = 128
PW = 48
NC = 2
NS = 16
GCH = 120
EPW = E // (NC * NS)
SPW = E // NS
NZ = 1816

K_CHUNKS = 2

EDGE_BLOCK = 3840
PRE_BLOCK = 2904
LAT_ROWS = 121
ROW_LEN = 240

_MESH = plsc.VectorSubcoreMesh(core_axis_name="c", subcore_axis_name="s")


def _pre_body(h, u, v, W1a, W1b, eb1, t1, t2):
    h_ = h[...]
    pad = jnp.zeros((h_.shape[0], TW - 86), jnp.float32)
    a = jnp.dot(h_, W1a[...], preferred_element_type=jnp.float32) + eb1[...]
    b = jnp.dot(h_, W1b[...], preferred_element_type=jnp.float32)
    t1[...] = jnp.concatenate([a, u[...], v[...], pad], axis=1)
    t2[...] = jnp.concatenate([b, u[...], v[...], pad], axis=1)


def _sc_gather(t1, t2, rowc, colc, n_edges):
    epw = n_edges // (NC * NS)

    @functools.partial(
        pl.kernel,
        out_type=[jax.ShapeDtypeStruct((n_edges, TW), jnp.float32),
                  jax.ShapeDtypeStruct((n_edges, TW), jnp.float32)],
        mesh=_MESH,
        scratch_types=[pltpu.VMEM((GCH,), jnp.int32),
                       pltpu.VMEM((GCH,), jnp.int32),
                       pltpu.VMEM((GCH,), jnp.int32),
                       pltpu.VMEM((GCH,), jnp.int32),
                       pltpu.VMEM((GCH, TW), jnp.float32),
                       pltpu.VMEM((GCH, TW), jnp.float32),
                       pltpu.VMEM((GCH, TW), jnp.float32),
                       pltpu.VMEM((GCH, TW), jnp.float32),
                       pltpu.SemaphoreType.DMA,
                       pltpu.SemaphoreType.DMA,
                       pltpu.SemaphoreType.DMA,
                       pltpu.SemaphoreType.DMA,
                       pltpu.SemaphoreType.DMA,
                       pltpu.SemaphoreType.DMA,
                       pltpu.SemaphoreType.DMA,
                       pltpu.SemaphoreType.DMA],
    )
    def k(t1_hbm, t2_hbm, row_hbm, col_hbm, g1_hbm, g2_hbm,
          i1a, i2a, i1b, i2b, b1a, b2a, b1b, b2b,
          s1, s2, s3, s4, s5, s6, s7, s8):
        wid = lax.axis_index("s") * NC + lax.axis_index("c")
        base = wid * epw

        @pl.loop(0, epw // (2 * GCH))
        def _(j):
            offa = base + 2 * j * GCH
            offb = offa + GCH
            pltpu.sync_copy(row_hbm.at[pl.ds(offa, GCH)], i1a)
            pltpu.sync_copy(col_hbm.at[pl.ds(offa, GCH)], i2a)
            ga1 = pltpu.async_copy(t1_hbm.at[i1a], b1a, s1)
            ga2 = pltpu.async_copy(t2_hbm.at[i2a], b2a, s2)
            pltpu.sync_copy(row_hbm.at[pl.ds(offb, GCH)], i1b)
            pltpu.sync_copy(col_hbm.at[pl.ds(offb, GCH)], i2b)
            gb1 = pltpu.async_copy(t1_hbm.at[i1b], b1b, s3)
            gb2 = pltpu.async_copy(t2_hbm.at[i2b], b2b, s4)
            ga1.wait()
            ga2.wait()
            wa1 = pltpu.async_copy(b1a, g1_hbm.at[pl.ds(offa, GCH)], s5)
            wa2 = pltpu.async_copy(b2a, g2_hbm.at[pl.ds(offa, GCH)], s6)
            gb1.wait()
            gb2.wait()
            wb1 = pltpu.async_copy(b1b, g1_hbm.at[pl.ds(offb, GCH)], s7)
            wb2 = pltpu.async_copy(b2b, g2_hbm.at[pl.ds(offb, GCH)], s8)
            wa1.wait()
            wa2.wait()
            wb1.wait()
            wb2.wait()

        if (epw // GCH) % 2 == 1:
            off = base + (epw // GCH - 1) * GCH
            pltpu.sync_copy(row_hbm.at[pl.ds(off, GCH)], i1a)
            pltpu.sync_copy(col_hbm.at[pl.ds(off, GCH)], i2a)
            g1 = pltpu.async_copy(t1_hbm.at[i1a], b1a, s1)
            g2 = pltpu.async_copy(t2_hbm.at[i2a], b2a, s2)
            g1.wait()
            g2.wait()
            pltpu.sync_copy(b1a, g1_hbm.at[pl.ds(off, GCH)])
            pltpu.sync_copy(b2a, g2_hbm.at[pl.ds(off, GCH)])

    return k(t1, t2, rowc, colc)


def _edge_body(g1, g2, ea, Ww, Wa, eW2, eb2, cW1, cb1, cW2, p):
    g1_ = g1[...]
    g2_ = g2[...]
    ur = g1_[:, 64:75]
    vr = g1_[:, 75:86]
    uc = g2_[:, 64:75]
    vc = g2_[:, 75:86]
    cs = jnp.sqrt(uc * uc + vc * vc)
    rs = jnp.sqrt(ur * ur + vr * vr)
    rd = (uc * ur + vc * vr) / (cs * rs)
    bf = jnp.bfloat16
    wdiff = jnp.concatenate([rd, cs, rs], axis=1).astype(bf)
    pre = (g1_[:, :64] + g2_[:, :64]
           + jnp.dot(wdiff, Ww[...].astype(bf), preferred_element_type=jnp.float32)
           + ea[...] * Wa[...])
    hid = jax.nn.relu(pre).astype(bf)
    ef = jax.nn.relu(jnp.dot(hid, eW2[...].astype(bf), preferred_element_type=jnp.float32) + eb2[...])
    ch = jax.nn.relu(jnp.dot(ef.astype(bf), cW1[...].astype(bf), preferred_element_type=jnp.float32) + cb1[...]).astype(bf)
    cf = jnp.dot(ch, cW2[...].astype(bf), preferred_element_type=jnp.float32)
    wu = cf[:, :11] * uc
    wv = cf[:, 11:] * vc
    ones = jnp.ones((g1_.shape[0], 1), jnp.float32)
    zpad = jnp.zeros((g1_.shape[0], TW - 87), jnp.float32)
    p[...] = jnp.concatenate([ef, wu, wv, ones, zpad], axis=1)


NH = N // NC
ACC_ROWS = NH + 8
SCH = 40
ZR = 1816


def _sc_scatter(p, row0c, row1c, o_prev, n_edges):
    epw = n_edges // NS

    @functools.partial(
        pl.kernel,
        out_type=jax.ShapeDtypeStruct((N, TW), jnp.float32),
        mesh=_MESH,
        scratch_types=[pltpu.VMEM((SCH,), jnp.int32),
                       pltpu.VMEM((SCH,), jnp.int32),
                       pltpu.VMEM((SCH, TW), jnp.float32),
                       pltpu.VMEM((SCH, TW), jnp.float32),
                       pltpu.VMEM_SHARED((ACC_ROWS, TW), jnp.float32),
                       pltpu.SemaphoreType.DMA,
                       pltpu.SemaphoreType.DMA,
                       pltpu.SemaphoreType.DMA,
                       pltpu.SemaphoreType.DMA],

    )
    def k(p_hbm, row0_hbm, row1_hbm, oprev_hbm, o_hbm,
          ia, ib, ba, bb, acc, s1, s2, s3, s4):
        c = lax.axis_index("c")
        s = lax.axis_index("s")
        c_lo = c * NH

        @pl.when(s < 7)
        def _():
            pltpu.sync_copy(oprev_hbm.at[pl.ds(c_lo + s * ZR, ZR)],
                            acc.at[pl.ds(s * ZR, ZR)])

        @pl.when(s == 7)
        def _():
            pltpu.sync_copy(oprev_hbm.at[pl.ds(c_lo + 7 * ZR, NH - 7 * ZR)],
                            acc.at[pl.ds(7 * ZR, NH - 7 * ZR)])

        plsc.subcore_barrier()
        base = s * epw

        def chunk_loop(row_hbm):
            @pl.loop(0, epw // (2 * SCH))
            def _(j):
                offa = base + 2 * j * SCH
                offb = offa + SCH
                la = pltpu.async_copy(row_hbm.at[pl.ds(offa, SCH)], ia, s1)
                da = pltpu.async_copy(p_hbm.at[pl.ds(offa, SCH)], ba, s2)
                lb = pltpu.async_copy(row_hbm.at[pl.ds(offb, SCH)], ib, s3)
                db = pltpu.async_copy(p_hbm.at[pl.ds(offb, SCH)], bb, s4)
                la.wait()
                da.wait()
                pltpu.sync_copy(ba, acc.at[ia], add=True)
                lb.wait()
                db.wait()
                pltpu.sync_copy(bb, acc.at[ib], add=True)

        @pl.when(c == 0)
        def _():
            chunk_loop(row0_hbm)

        @pl.when(c == 1)
        def _():
            chunk_loop(row1_hbm)

        plsc.subcore_barrier()

        @pl.when(s < 7)
        def _():
            pltpu.sync_copy(acc.at[pl.ds(s * ZR, ZR)],
                            o_hbm.at[pl.ds(c_lo + s * ZR, ZR)])

        @pl.when(s == 7)
        def _():
            pltpu.sync_copy(acc.at[pl.ds(7 * ZR, NH - 7 * ZR)],
                            o_hbm.at[pl.ds(c_lo + 7 * ZR, NH - 7 * ZR)])

    return k(p, row0c, row1c, o_prev)


def _node_body(h, o, nW1, nb1, nW2, nb2, h_out, u_out, v_out):
    h_ = h[...]
    o_ = o[...]
    agg = o_[:, :64]
    sums = o_[:, 64:86]
    cnt = jnp.maximum(o_[:, 86:87], 1.0)
    mean = jnp.clip(sums / cnt, -100.0, 100.0)
    u_out[...] = mean[:, :11]
    v_out[...] = mean[:, 11:]
    lat = jnp.mean(agg, axis=0, keepdims=True)
    cat = jnp.concatenate([h_, agg, jnp.broadcast_to(lat, agg.shape)], axis=1)
    hid = jax.nn.relu(jnp.dot(cat, nW1[...], preferred_element_type=jnp.float32) + nb1[...])
    h_out[...] = jnp.dot(hid, nW2[...], preferred_element_type=jnp.float32) + nb2[...] + h_


@jax.jit
def kernel(h, edge_index, u, v, edge_attr, eW1, eb1, eW2, eb2,
           nW1, nb1, nW2, nb2, cW1, cb1, cW2):
    row = edge_index[0]
    col = edge_index[1]
    W1a = eW1[0:64]
    W1b = eW1[64:128]
    Ww = eW1[128:161]
    Wa = eW1[161:162]

    wb = lambda a: pl.BlockSpec(a.shape, lambda i: (0,) * a.ndim)
    pb = lambda d: pl.BlockSpec((PRE_BLOCK, d), lambda i: (i, 0))
    t1, t2 = pl.pallas_call(
        _pre_body,
        grid=(N // PRE_BLOCK,),
        in_specs=[pb(D), pb(11), pb(11), wb(W1a), wb(W1b), wb(eb1)],
        out_specs=[pb(TW), pb(TW)],
        out_shape=[jax.ShapeDtypeStruct((N, TW), jnp.float32),
                   jax.ShapeDtypeStruct((N, TW), jnp.float32)],
    )(h, u, v, W1a, W1b, eb1)

    sink = NH + (jnp.arange(E, dtype=jnp.int32) & 7)
    row0 = jnp.where(row < NH, row, sink)
    row1 = jnp.where(row >= NH, row - NH, sink)

    ebk = lambda d: pl.BlockSpec((EDGE_BLOCK, d), lambda i: (i, 0))
    eh = E // K_CHUNKS
    o = jnp.zeros((N, TW), jnp.float32)
    for kc in range(K_CHUNKS):
        sl = slice(kc * eh, (kc + 1) * eh)
        g1, g2 = _sc_gather(t1, t2, row[sl], col[sl], eh)
        p = pl.pallas_call(
            _edge_body,
            grid=(eh // EDGE_BLOCK,),
            in_specs=[ebk(TW), ebk(TW), ebk(1),
                      wb(Ww), wb(Wa), wb(eW2), wb(eb2), wb(cW1), wb(cb1), wb(cW2)],
            out_specs=[ebk(TW)],
            out_shape=[jax.ShapeDtypeStruct((eh, TW), jnp.float32)],
        )(g1, g2, edge_attr[sl], Ww, Wa, eW2, eb2, cW1, cb1, cW2)[0]
        o = _sc_scatter(p, row0[sl], row1[sl], o, eh)

    nbk = lambda d: pl.BlockSpec((ROW_LEN, d), lambda i: (i, 0))
    h_out, agg_u, agg_v = pl.pallas_call(
        _node_body,
        grid=(LAT_ROWS,),
        in_specs=[nbk(D), nbk(TW),
                  wb(nW1), wb(nb1), wb(nW2), wb(nb2)],
        out_specs=[nbk(D), nbk(11), nbk(11)],
        out_shape=[jax.ShapeDtypeStruct((N, D), jnp.float32),
                   jax.ShapeDtypeStruct((N, 11), jnp.float32),
                   jax.ShapeDtypeStruct((N, 11), jnp.float32)],
    )(h, o, nW1, nb1, nW2, nb2)
    return (h_out, agg_u, agg_v)
